# baseline (speedup 1.0000x reference)
import functools

import jax
import jax.numpy as jnp
from jax.experimental import pallas as pl
from jax.experimental.pallas import tpu as pltpu

_ALPHA = 0.9
_NUM_PARTS = 7
_FEATURE_DIM = 256


def _fused_kernel(vis_ref, mask_ref, hbm_ref, w_ref, b_ref, out_ref,
                  buf_ref, sem_ref, *, T, V, D, FD, alpha, G, NBUF, JB):
    f32 = jnp.float32
    i = pl.program_id(0)
    n = pl.num_programs(0)

    def _start(block, slot):
        bb = block // JB
        jj = block % JB
        pltpu.make_async_copy(
            hbm_ref.at[bb, pl.ds(jj * G, G)],
            buf_ref.at[slot],
            sem_ref.at[slot],
        ).start()

    @pl.when(i == 0)
    def _prologue():
        for k in range(NBUF - 1):
            _start(k, k)

    nxt = i + NBUF - 1

    @pl.when(nxt < n)
    def _refill():
        _start(nxt, jax.lax.rem(nxt, NBUF))

    slot = jax.lax.rem(i, NBUF)
    pltpu.make_async_copy(
        hbm_ref.at[0, pl.ds(0, G)], buf_ref.at[slot], sem_ref.at[slot]
    ).wait()

    v = jnp.zeros((G, V), f32)
    A = [None] * T
    C = [None] * T
    for t in range(T - 1, -1, -1):
        vis_t = vis_ref[0, :, t, :]
        m = mask_ref[0, :, t:t + 1]
        v_nz = (v != 0.0).astype(f32)
        d_nz = (vis_t != 0.0).astype(f32)
        xor = v_nz + d_nz - 2.0 * v_nz * d_nz
        a_t = v * vis_t * alpha + xor * v
        c_t = v * vis_t * (1.0 - alpha) + xor * vis_t
        A[t] = m * a_t + (1.0 - m)
        C[t] = m * c_t
        v = m * jnp.maximum(v, vis_t) + (1.0 - m) * v

    part_row = jax.lax.broadcasted_iota(jnp.int32, (V, D), 0)
    part_col = jax.lax.broadcasted_iota(jnp.int32, (V, D), 1) // FD
    expand = (part_row == part_col).astype(f32)

    acc = jnp.zeros((G, D), f32)
    prod = jnp.ones((G, V), f32)
    for t in range(T):
        w_t = C[t] * prod
        prod = prod * A[t]
        w_full = jax.lax.dot_general(
            w_t, expand, (((1,), (0,)), ((), ())),
            preferred_element_type=f32)
        acc = acc + w_full * buf_ref[slot, :, t, :]

    lin = jax.lax.dot_general(
        acc, w_ref[:, :], (((1,), (1,)), ((), ())),
        preferred_element_type=f32)
    lin = lin + b_ref[:, :]
    new_mask = jnp.max(mask_ref[0, :, :], axis=1, keepdims=True)
    out_ref[0, :, :] = jnp.where(new_mask > 0.0, lin, 0.0)


def kernel(embs, vis, masks, W, b):
    B, N, T, D = embs.shape
    V = vis.shape[-1]
    K = W.shape[0]
    FD = D // _NUM_PARTS
    G = 32
    NBUF = 6
    JB = N // G

    masks2 = masks.astype(jnp.float32)
    b2 = b.reshape(1, K)

    body = functools.partial(_fused_kernel, T=T, V=V, D=D, FD=FD,
                             alpha=_ALPHA, G=G, NBUF=NBUF, JB=JB)
    out = pl.pallas_call(
        body,
        grid=(B * JB,),
        in_specs=[
            pl.BlockSpec((1, G, T, V), lambda i: (i // (N // G), i % (N // G), 0, 0)),
            pl.BlockSpec((1, G, T), lambda i: (i // (N // G), i % (N // G), 0)),
            pl.BlockSpec(memory_space=pl.ANY),
            pl.BlockSpec((K, D), lambda i: (0, 0)),
            pl.BlockSpec((1, K), lambda i: (0, 0)),
        ],
        out_specs=pl.BlockSpec((1, G, K), lambda i: (i // (N // G), i % (N // G), 0)),
        out_shape=jax.ShapeDtypeStruct((B, N, K), jnp.float32),
        scratch_shapes=[
            pltpu.VMEM((NBUF, G, T, D), jnp.float32),
            pltpu.SemaphoreType.DMA((NBUF,)),
        ],
    )(vis, masks2, embs, W, b2)
    return out


# speedup vs baseline: 1.1085x; 1.1085x over previous
import functools

import jax
import jax.numpy as jnp
from jax.experimental import pallas as pl
from jax.experimental.pallas import tpu as pltpu

_ALPHA = 0.9
_NUM_PARTS = 7
_FEATURE_DIM = 256


def _fused_kernel(vis_ref, mask_ref, hbm_ref, w_ref, b_ref, out_ref,
                  buf_ref, sem_ref, *, T, V, D, FD, alpha, G, NBUF):
    f32 = jnp.float32
    i = pl.program_id(0)
    n = pl.num_programs(0)

    def _start(block, slot, t):
        pltpu.make_async_copy(
            hbm_ref.at[pl.ds(block * G, G), t],
            buf_ref.at[slot, t],
            sem_ref.at[slot, t],
        ).start()

    @pl.when(i == 0)
    def _prologue():
        for k in range(NBUF - 1):
            for t in range(T):
                _start(k, k, t)

    nxt = i + NBUF - 1

    @pl.when(nxt < n)
    def _refill():
        nslot = jax.lax.rem(nxt, NBUF)
        for t in range(T):
            _start(nxt, nslot, t)

    slot = jax.lax.rem(i, NBUF)

    v = jnp.zeros((G, V), f32)
    A = [None] * T
    C = [None] * T
    for t in range(T - 1, -1, -1):
        vis_t = vis_ref[:, t, :]
        m = mask_ref[:, t:t + 1]
        v_nz = (v != 0.0).astype(f32)
        d_nz = (vis_t != 0.0).astype(f32)
        xor = v_nz + d_nz - 2.0 * v_nz * d_nz
        a_t = v * vis_t * alpha + xor * v
        c_t = v * vis_t * (1.0 - alpha) + xor * vis_t
        A[t] = m * a_t + (1.0 - m)
        C[t] = m * c_t
        v = m * jnp.maximum(v, vis_t) + (1.0 - m) * v

    part_row = jax.lax.broadcasted_iota(jnp.int32, (V, D), 0)
    part_col = jax.lax.broadcasted_iota(jnp.int32, (V, D), 1) // FD
    expand = (part_row == part_col).astype(f32)

    acc = jnp.zeros((G, D), f32)
    prod = jnp.ones((G, V), f32)
    for t in range(T):
        w_t = C[t] * prod
        prod = prod * A[t]
        w_full = jax.lax.dot_general(
            w_t, expand, (((1,), (0,)), ((), ())),
            preferred_element_type=f32)
        pltpu.make_async_copy(
            hbm_ref.at[pl.ds(0, G), t], buf_ref.at[slot, t],
            sem_ref.at[slot, t]).wait()
        acc = acc + w_full * buf_ref[slot, t]

    lin = jax.lax.dot_general(
        acc, w_ref[:, :], (((1,), (1,)), ((), ())),
        preferred_element_type=f32)
    lin = lin + b_ref[:, :]
    new_mask = jnp.max(mask_ref[:, :], axis=1, keepdims=True)
    out_ref[:, :] = jnp.where(new_mask > 0.0, lin, 0.0)


def kernel(embs, vis, masks, W, b):
    B, N, T, D = embs.shape
    V = vis.shape[-1]
    K = W.shape[0]
    FD = D // _NUM_PARTS
    R = B * N
    G = 32
    NBUF = 4

    embs3 = embs.reshape(R, T, D)
    vis3 = vis.reshape(R, T, V)
    masks2 = masks.reshape(R, T).astype(jnp.float32)
    b2 = b.reshape(1, K)

    body = functools.partial(_fused_kernel, T=T, V=V, D=D, FD=FD,
                             alpha=_ALPHA, G=G, NBUF=NBUF)
    out = pl.pallas_call(
        body,
        grid=(R // G,),
        in_specs=[
            pl.BlockSpec((G, T, V), lambda i: (i, 0, 0)),
            pl.BlockSpec((G, T), lambda i: (i, 0)),
            pl.BlockSpec(memory_space=pl.ANY),
            pl.BlockSpec((K, D), lambda i: (0, 0)),
            pl.BlockSpec((1, K), lambda i: (0, 0)),
        ],
        out_specs=pl.BlockSpec((G, K), lambda i: (i, 0)),
        out_shape=jax.ShapeDtypeStruct((R, K), jnp.float32),
        scratch_shapes=[
            pltpu.VMEM((NBUF, T, G, D), jnp.float32),
            pltpu.SemaphoreType.DMA((NBUF, T)),
        ],
    )(vis3, masks2, embs3, W, b2)
    return out.reshape(B, N, K)


# speedup vs baseline: 1.3031x; 1.1755x over previous
import functools

import jax
import jax.numpy as jnp
from jax.experimental import pallas as pl

_ALPHA = 0.9
_NUM_PARTS = 7
_FEATURE_DIM = 256


def _weights(vis_ref, mask_ref, r0, G, T, V):
    f32 = jnp.float32
    v = jnp.zeros((G, V), f32)
    A = [None] * T
    C = [None] * T
    for t in range(T - 1, -1, -1):
        vis_t = vis_ref[r0:r0 + G, t, :]
        m = mask_ref[r0:r0 + G, t:t + 1]
        v_nz = (v != 0.0).astype(f32)
        d_nz = (vis_t != 0.0).astype(f32)
        xor = v_nz + d_nz - 2.0 * v_nz * d_nz
        a_t = v * vis_t * _ALPHA + xor * v
        c_t = v * vis_t * (1.0 - _ALPHA) + xor * vis_t
        A[t] = m * a_t + (1.0 - m)
        C[t] = m * c_t
        v = m * jnp.maximum(v, vis_t) + (1.0 - m) * v
    W = [None] * T
    prod = jnp.ones((G, V), f32)
    for t in range(T):
        W[t] = C[t] * prod
        prod = prod * A[t]
    return W


def _fused_kernel(vis_ref, mask_ref, e1_ref, e2_ref, w_ref, b_ref, out_ref,
                  *, T, V, D, FD, G):
    f32 = jnp.float32

    part_row = jax.lax.broadcasted_iota(jnp.int32, (V, D), 0)
    part_col = jax.lax.broadcasted_iota(jnp.int32, (V, D), 1) // FD
    expand = (part_row == part_col).astype(f32)

    for half, e_ref in ((0, e1_ref), (1, e2_ref)):
        r0 = half * G
        wts = _weights(vis_ref, mask_ref, r0, G, T, V)
        acc = jnp.zeros((G, D), f32)
        for t in range(T):
            w_full = jax.lax.dot_general(
                wts[t], expand, (((1,), (0,)), ((), ())),
                preferred_element_type=f32)
            acc = acc + w_full * e_ref[:, t, :]
        lin = jax.lax.dot_general(
            acc, w_ref[:, :], (((1,), (1,)), ((), ())),
            preferred_element_type=f32)
        lin = lin + b_ref[:, :]
        new_mask = jnp.max(mask_ref[r0:r0 + G, :], axis=1, keepdims=True)
        out_ref[r0:r0 + G, :] = jnp.where(new_mask > 0.0, lin, 0.0)


def kernel(embs, vis, masks, W, b):
    B, N, T, D = embs.shape
    V = vis.shape[-1]
    K = W.shape[0]
    FD = D // _NUM_PARTS
    R = B * N
    G = 32

    embs3 = embs.reshape(R, T, D)
    vis3 = vis.reshape(R, T, V)
    masks2 = masks.reshape(R, T).astype(jnp.float32)
    b2 = b.reshape(1, K)

    body = functools.partial(_fused_kernel, T=T, V=V, D=D, FD=FD, G=G)
    out = pl.pallas_call(
        body,
        grid=(R // (2 * G),),
        in_specs=[
            pl.BlockSpec((2 * G, T, V), lambda i: (i, 0, 0)),
            pl.BlockSpec((2 * G, T), lambda i: (i, 0)),
            pl.BlockSpec((G, T, D), lambda i: (2 * i, 0, 0)),
            pl.BlockSpec((G, T, D), lambda i: (2 * i + 1, 0, 0)),
            pl.BlockSpec((K, D), lambda i: (0, 0)),
            pl.BlockSpec((1, K), lambda i: (0, 0)),
        ],
        out_specs=pl.BlockSpec((2 * G, K), lambda i: (i, 0)),
        out_shape=jax.ShapeDtypeStruct((R, K), jnp.float32),
    )(vis3, masks2, embs3, embs3, W, b2)
    return out.reshape(B, N, K)
